# Initial kernel scaffold; baseline (speedup 1.0000x reference)
import jax
import jax.numpy as jnp
from jax.experimental import pallas as pl


def kernel(pred_o, target_o):
    raise NotImplementedError("write your pallas kernel here")


# speedup vs baseline: 2.6556x; 2.6556x over previous
import functools

import jax
import jax.numpy as jnp
from jax.experimental import pallas as pl
from jax.experimental.pallas import tpu as pltpu

_B, _C, _F = 1024, 2, 4096
_BB = 64


def _tc_body(p_ref, t_ref, o_ref):
    i = pl.program_id(0)
    ps = p_ref[:, 0, :]
    pc = p_ref[:, 1, :]
    ts = t_ref[:, 0, :]
    tc = t_ref[:, 1, :]
    m = tc != 0.0
    term = jnp.where(m, (ps - ts) ** 2 + (pc - tc) ** 2, 0.0)
    partial = jnp.sum(term)

    @pl.when(i == 0)
    def _():
        o_ref[0, 0] = 0.0

    o_ref[0, 0] += partial


def _tc_loss(pred_o, target_o):
    nb = pred_o.shape[0]
    grid = nb // _BB
    out = pl.pallas_call(
        _tc_body,
        grid=(grid,),
        in_specs=[
            pl.BlockSpec((_BB, _C, _F), lambda i: (i, 0, 0)),
            pl.BlockSpec((_BB, _C, _F), lambda i: (i, 0, 0)),
        ],
        out_specs=pl.BlockSpec(memory_space=pltpu.SMEM),
        out_shape=jax.ShapeDtypeStruct((1, 1), jnp.float32),
    )(pred_o, target_o)
    return out[0, 0]


def kernel(pred_o, target_o):
    return _tc_loss(pred_o, target_o)
